# baseline (speedup 1.0000x reference)
import jax
import jax.numpy as jnp
from jax import lax
from jax.experimental import pallas as pl
from jax.experimental.pallas import tpu as pltpu
from jax.experimental.pallas import tpu_sc as plsc

CYCLE = 168
T = 200
D = 64
B = 4096
NC = 2
NS = 16
NW = NC * NS
BPW = B // NW
LANES = 16
NGRP = BPW // LANES
TABW = CYCLE * D


def _body(idx_hbm, data_hbm, off_hbm, out_hbm, tab_v, idx_v, off_v, stage_v, sem):
    w = lax.axis_index("s") * NC + lax.axis_index("c")
    base = w * BPW

    pltpu.sync_copy(data_hbm, tab_v)
    pltpu.sync_copy(idx_hbm.at[pl.ds(base, BPW)], idx_v)
    pltpu.sync_copy(off_hbm, off_v)

    offv = off_v[...]
    rb0 = []
    for j in range(NGRP):
        v = idx_v[pl.ds(j * LANES, LANES)]
        v = lax.rem(v + offv, jnp.int32(CYCLE))
        v = jnp.where(v < 0, v + jnp.int32(CYCLE), v)
        rb0.append(v)

    def emit_t(t, k, rb):
        @pl.when(t >= 2)
        def _():
            pltpu.make_async_copy(
                stage_v.at[k], out_hbm.at[0, :, w], sem
            ).wait()

        @plsc.parallel_loop(0, D // 16, unroll=2)
        def _(ct):
            cb = ct * jnp.int32(16 * CYCLE)
            for ci in range(16):
                for j in range(NGRP):
                    a = rb[j] + (cb + jnp.int32(ci * CYCLE))
                    vals = plsc.load_gather(tab_v, [a])
                    stage_v[k, (ct * 2 + ci // 8), ci % 8, pl.ds(j * LANES, LANES)] = vals

        pltpu.make_async_copy(stage_v.at[k], out_hbm.at[t, :, w], sem).start()

        out = []
        for j in range(NGRP):
            nv = rb[j] + jnp.int32(1)
            out.append(jnp.where(nv == jnp.int32(CYCLE), jnp.int32(0), nv))
        return out

    def pair(i, rb):
        rb = tuple(rb)
        t0 = i * 2
        rb = emit_t(t0, 0, rb)
        rb = emit_t(t0 + 1, 1, rb)
        return tuple(rb)

    lax.fori_loop(0, T // 2, pair, tuple(rb0))

    for k in range(2):
        pltpu.make_async_copy(stage_v.at[k], out_hbm.at[0, :, w], sem).wait()


@jax.jit
def _run(index, data, length):
    off = jnp.full((LANES,), 1, dtype=jnp.int32) * (
        jnp.asarray(length, dtype=jnp.int32) - jnp.int32(T)
    )
    mesh = plsc.VectorSubcoreMesh(core_axis_name="c", subcore_axis_name="s")
    out5 = pl.kernel(
        _body,
        out_type=jax.ShapeDtypeStruct((T, D // 8, NW, 8, BPW), jnp.float32),
        mesh=mesh,
        compiler_params=pltpu.CompilerParams(needs_layout_passes=False),
        scratch_types=[
            pltpu.VMEM((TABW,), jnp.float32),
            pltpu.VMEM((BPW,), jnp.int32),
            pltpu.VMEM((LANES,), jnp.int32),
            pltpu.VMEM((2, D // 8, 8, BPW), jnp.float32),
            pltpu.SemaphoreType.DMA,
        ],
    )(index, jnp.transpose(data).reshape(TABW), off)
    return jnp.transpose(out5, (2, 4, 0, 1, 3)).reshape(B, T, D)


def kernel(index, length, data):
    return _run(index, data, length)


# speedup vs baseline: 1.9123x; 1.9123x over previous
import jax
import jax.numpy as jnp
from jax import lax
from jax.experimental import pallas as pl
from jax.experimental.pallas import tpu as pltpu
from jax.experimental.pallas import tpu_sc as plsc

CYCLE = 168
T = 200
D = 64
B = 4096
NC = 2
NS = 16
NW = NC * NS
BPW = B // NW
LANES = 16
NGRP = BPW // LANES
TABW = CYCLE * D


def _body(idx_hbm, data_hbm, off_hbm, out_hbm, tab_v, idx_v, off_v, stage_v, sem):
    w = lax.axis_index("s") * NC + lax.axis_index("c")
    base = w * BPW

    pltpu.sync_copy(data_hbm, tab_v)
    pltpu.sync_copy(idx_hbm.at[pl.ds(base, BPW)], idx_v)
    pltpu.sync_copy(off_hbm, off_v)

    offv = off_v[...]
    rb0 = []
    for j in range(NGRP):
        v = idx_v[pl.ds(j * LANES, LANES)]
        v = lax.rem(v + offv, jnp.int32(CYCLE))
        v = jnp.where(v < 0, v + jnp.int32(CYCLE), v)
        rb0.append(v)

    def emit_t(t, k, rb):
        @pl.when(t >= 2)
        def _():
            pltpu.make_async_copy(
                stage_v.at[k], out_hbm.at[0, :, w], sem
            ).wait()

        @plsc.parallel_loop(0, D // 8, unroll=2)
        def _(ct):
            cb = ct * jnp.int32(8 * CYCLE)
            for ci in range(8):
                for j in range(NGRP):
                    a = rb[j] + (cb + jnp.int32(ci * CYCLE))
                    vals = plsc.load_gather(tab_v, [a])
                    stage_v[k, ct, ci, pl.ds(j * LANES, LANES)] = vals

        pltpu.make_async_copy(stage_v.at[k], out_hbm.at[t, :, w], sem).start()

        out = []
        for j in range(NGRP):
            nv = rb[j] + jnp.int32(1)
            out.append(jnp.where(nv == jnp.int32(CYCLE), jnp.int32(0), nv))
        return out

    def pair(i, rb):
        rb = tuple(rb)
        t0 = i * 2
        rb = emit_t(t0, 0, rb)
        rb = emit_t(t0 + 1, 1, rb)
        return tuple(rb)

    lax.fori_loop(0, T // 2, pair, tuple(rb0))

    for k in range(2):
        pltpu.make_async_copy(stage_v.at[k], out_hbm.at[0, :, w], sem).wait()


@jax.jit
def _run(index, data, length):
    off = jnp.full((LANES,), 1, dtype=jnp.int32) * (
        jnp.asarray(length, dtype=jnp.int32) - jnp.int32(T)
    )
    mesh = plsc.VectorSubcoreMesh(core_axis_name="c", subcore_axis_name="s")
    out5 = pl.kernel(
        _body,
        out_type=jax.ShapeDtypeStruct((T, D // 8, NW, 8, BPW), jnp.float32),
        mesh=mesh,
        compiler_params=pltpu.CompilerParams(needs_layout_passes=False),
        scratch_types=[
            pltpu.VMEM((TABW,), jnp.float32),
            pltpu.VMEM((BPW,), jnp.int32),
            pltpu.VMEM((LANES,), jnp.int32),
            pltpu.VMEM((2, D // 8, 8, BPW), jnp.float32),
            pltpu.SemaphoreType.DMA,
        ],
    )(index, jnp.transpose(data).reshape(TABW), off)
    return jnp.transpose(out5, (2, 4, 0, 1, 3)).reshape(B, T, D)


def kernel(index, length, data):
    return _run(index, data, length)
